# baseline (speedup 1.0000x reference)
import functools

import jax
import jax.numpy as jnp
from jax import lax
from jax.experimental import pallas as pl
from jax.experimental.pallas import tpu as pltpu
from jax.experimental.pallas import tpu_sc as plsc

_N = 10000
_NPAD = 10240
_E = 320000
_NTILES = 32
_K = 125
_EPT = _E // _NTILES
_CHUNKS = _EPT // _K
_RPT = _NPAD // 16
_RDEC = 400
_NB = 8


def _mm1_body(x_ref, w_ref, b_ref, o_ref):
    o_ref[...] = (
        jnp.dot(x_ref[...], w_ref[...], preferred_element_type=jnp.float32)
        + b_ref[...]
    )


def _mm1(x, w, b):
    n, din = x.shape
    dout = w.shape[1]
    return pl.pallas_call(
        _mm1_body,
        out_shape=jax.ShapeDtypeStruct((n, dout), jnp.float32),
    )(x, w, b)


def _mm2_body(p_ref, w_ref, b_ref, o_ref):
    h = jnp.maximum(p_ref[0] + p_ref[1], 0.0)
    o_ref[...] = (
        jnp.dot(h, w_ref[...], preferred_element_type=jnp.float32) + b_ref[...]
    )


def _mm2(p, w, b):
    n = p.shape[1]
    dout = w.shape[1]
    return pl.pallas_call(
        _mm2_body,
        out_shape=jax.ShapeDtypeStruct((n, dout), jnp.float32),
    )(p, w, b)


def _dec_body(zp_ref, zpb_ref, wd1_ref, bd1_ref, wd2_ref, bd2_ref,
              recon_ref, adj_ref):
    zfull = (zp_ref[0] + zp_ref[1])[:_N]
    zblk = zpb_ref[0] + zpb_ref[1]
    d = jnp.maximum(
        jnp.dot(zblk, wd1_ref[...], preferred_element_type=jnp.float32)
        + bd1_ref[...], 0.0)
    recon_ref[...] = jnp.maximum(
        jnp.dot(d, wd2_ref[...], preferred_element_type=jnp.float32)
        + bd2_ref[...], 0.0)
    logits = lax.dot_general(zblk, zfull, (((1,), (1,)), ((), ())),
                             preferred_element_type=jnp.float32)
    adj_ref[...] = 0.5 * jnp.tanh(0.5 * logits) + 0.5


def _decode(zp, wd1, bd1, wd2, bd2):
    n = _N
    npad = zp.shape[1]
    dz = zp.shape[2]
    d1 = wd1.shape[1]
    d0 = wd2.shape[1]
    grid = (n // _RDEC,)
    return pl.pallas_call(
        _dec_body,
        grid=grid,
        in_specs=[
            pl.BlockSpec((2, npad, dz), lambda i: (0, 0, 0)),
            pl.BlockSpec((2, _RDEC, dz), lambda i: (0, i, 0)),
            pl.BlockSpec((dz, d1), lambda i: (0, 0)),
            pl.BlockSpec((1, d1), lambda i: (0, 0)),
            pl.BlockSpec((d1, d0), lambda i: (0, 0)),
            pl.BlockSpec((1, d0), lambda i: (0, 0)),
        ],
        out_specs=[
            pl.BlockSpec((_RDEC, d0), lambda i: (i, 0)),
            pl.BlockSpec((_RDEC, n), lambda i: (i, 0)),
        ],
        out_shape=[
            jax.ShapeDtypeStruct((n, d0), jnp.float32),
            jax.ShapeDtypeStruct((n, n), jnp.float32),
        ],
    )(zp, zp, wd1, bd1, wd2, bd2)


def _make_segsum(d):
    mesh = plsc.VectorSubcoreMesh(core_axis_name="c", subcore_axis_name="s")

    @functools.partial(
        pl.kernel,
        out_type=jax.ShapeDtypeStruct((2, _NPAD, d), jnp.float32),
        mesh=mesh,
        scratch_types=[
            pltpu.VMEM((_CHUNKS, _K), jnp.int32),
            pltpu.VMEM((_CHUNKS, _K), jnp.int32),
          ] + [pltpu.VMEM((_K, d), jnp.float32)] * _NB
          + [pltpu.VMEM_SHARED((_NPAD, d), jnp.float32)]
          + [pltpu.SemaphoreType.DMA] * (2 * _NB),
        compiler_params=pltpu.CompilerParams(use_tc_tiling_on_sc=False),
    )
    def segsum(sup_hbm, src_hbm, dst_hbm, zero_hbm, out_hbm,
               src_v, dst_v, *bufs):
        rows = bufs[:_NB]
        acc_sh = bufs[_NB]
        gsems = bufs[_NB + 1:2 * _NB + 1]
        ssems = bufs[2 * _NB + 1:]
        cid = lax.axis_index("c")
        sid = lax.axis_index("s")
        tile = cid * 16 + sid
        r0 = sid * _RPT
        pltpu.sync_copy(zero_hbm.at[pl.ds(r0, _RPT)],
                        acc_sh.at[pl.ds(r0, _RPT)])
        c0 = tile * _CHUNKS
        pltpu.sync_copy(src_hbm.at[pl.ds(c0, _CHUNKS)], src_v)
        pltpu.sync_copy(dst_hbm.at[pl.ds(c0, _CHUNKS)], dst_v)
        plsc.subcore_barrier()

        nb = _NB
        gla = _NB // 2
        for b in range(gla):
            pltpu.async_copy(sup_hbm.at[src_v.at[b]], rows[b], gsems[b])

        def body(g, carry):
            for b in range(nb):
                i = g * nb + b
                b2 = (b + gla) % nb
                pltpu.make_async_copy(sup_hbm.at[src_v.at[i]],
                                      rows[b], gsems[b]).wait()
                pltpu.async_copy(rows[b], acc_sh.at[dst_v.at[i]], ssems[b],
                                 add=True)
                nxt = i + gla

                @pl.when(nxt < _CHUNKS)
                def _():
                    @pl.when(nxt >= nb)
                    def _():
                        pltpu.make_async_copy(
                            rows[b2], acc_sh.at[dst_v.at[0]],
                            ssems[b2]).wait()

                    pltpu.async_copy(sup_hbm.at[src_v.at[nxt]],
                                     rows[b2], gsems[b2])
            return carry

        lax.fori_loop(0, _CHUNKS // nb, body, 0)
        for b in range(nb):
            pltpu.make_async_copy(rows[b], acc_sh.at[dst_v.at[0]],
                                  ssems[b]).wait()
        plsc.subcore_barrier()
        pltpu.sync_copy(acc_sh.at[pl.ds(r0, _RPT)],
                        out_hbm.at[cid, pl.ds(r0, _RPT)])

    return segsum


_segsum64 = _make_segsum(64)
_segsum32 = _make_segsum(32)


def kernel(fea, edge_index, W1, b1, W2, b2, Wd1, bd1, Wd2, bd2):
    src = edge_index[0].reshape(_NTILES * _CHUNKS, _K)
    dst = edge_index[1].reshape(_NTILES * _CHUNKS, _K)
    zero64 = jnp.zeros((_NPAD, 64), jnp.float32)
    zero32 = jnp.zeros((_NPAD, 32), jnp.float32)

    sup1 = _mm1(fea, W1, b1.reshape(1, -1))
    p1 = _segsum64(sup1, src, dst, zero64)
    sup2 = _mm2(p1, W2, b2.reshape(1, -1))
    p2 = _segsum32(sup2, src, dst, zero32)
    recon, adj = _decode(p2, Wd1, bd1.reshape(1, -1), Wd2, bd2.reshape(1, -1))
    return recon, adj


# speedup vs baseline: 1.0002x; 1.0002x over previous
import functools

import jax
import jax.numpy as jnp
from jax import lax
from jax.experimental import pallas as pl
from jax.experimental.pallas import tpu as pltpu
from jax.experimental.pallas import tpu_sc as plsc

_N = 10000
_NPAD = 10240
_E = 320000
_NTILES = 32
_K = 125
_EPT = _E // _NTILES
_CHUNKS = _EPT // _K
_RPT = _NPAD // 16
_RDEC = 400
_NB = 8


def _mm1_body(x_ref, w_ref, b_ref, o_ref):
    o_ref[...] = (
        jnp.dot(x_ref[...], w_ref[...], preferred_element_type=jnp.float32)
        + b_ref[...]
    )


def _mm1(x, w, b):
    n, din = x.shape
    dout = w.shape[1]
    return pl.pallas_call(
        _mm1_body,
        out_shape=jax.ShapeDtypeStruct((n, dout), jnp.float32),
    )(x, w, b)


def _mm2_body(p_ref, w_ref, b_ref, o_ref):
    h = jnp.maximum(p_ref[0] + p_ref[1], 0.0)
    o_ref[...] = (
        jnp.dot(h, w_ref[...], preferred_element_type=jnp.float32) + b_ref[...]
    )


def _mm2(p, w, b):
    n = p.shape[1]
    dout = w.shape[1]
    return pl.pallas_call(
        _mm2_body,
        out_shape=jax.ShapeDtypeStruct((n, dout), jnp.float32),
    )(p, w, b)


def _dec_body(zp_ref, zpb_ref, wd1_ref, bd1_ref, wd2_ref, bd2_ref,
              recon_ref, adj_ref):
    zfull = (zp_ref[0] + zp_ref[1])[:_N]
    zblk = zpb_ref[0] + zpb_ref[1]
    d = jnp.maximum(
        jnp.dot(zblk, wd1_ref[...], preferred_element_type=jnp.float32)
        + bd1_ref[...], 0.0)
    recon_ref[...] = jnp.maximum(
        jnp.dot(d, wd2_ref[...], preferred_element_type=jnp.float32)
        + bd2_ref[...], 0.0)
    logits = lax.dot_general(zblk, zfull, (((1,), (1,)), ((), ())),
                             preferred_element_type=jnp.float32)
    adj_ref[...] = 0.5 * jnp.tanh(0.5 * logits) + 0.5


def _decode(zp, wd1, bd1, wd2, bd2):
    n = _N
    npad = zp.shape[1]
    dz = zp.shape[2]
    d1 = wd1.shape[1]
    d0 = wd2.shape[1]
    grid = (n // _RDEC,)
    return pl.pallas_call(
        _dec_body,
        grid=grid,
        in_specs=[
            pl.BlockSpec((2, npad, dz), lambda i: (0, 0, 0)),
            pl.BlockSpec((2, _RDEC, dz), lambda i: (0, i, 0)),
            pl.BlockSpec((dz, d1), lambda i: (0, 0)),
            pl.BlockSpec((1, d1), lambda i: (0, 0)),
            pl.BlockSpec((d1, d0), lambda i: (0, 0)),
            pl.BlockSpec((1, d0), lambda i: (0, 0)),
        ],
        out_specs=[
            pl.BlockSpec((_RDEC, d0), lambda i: (i, 0)),
            pl.BlockSpec((_RDEC, n), lambda i: (i, 0)),
        ],
        out_shape=[
            jax.ShapeDtypeStruct((n, d0), jnp.float32),
            jax.ShapeDtypeStruct((n, n), jnp.float32),
        ],
    )(zp, zp, wd1, bd1, wd2, bd2)


def _make_segsum(d):
    mesh = plsc.VectorSubcoreMesh(core_axis_name="c", subcore_axis_name="s")

    @functools.partial(
        pl.kernel,
        out_type=jax.ShapeDtypeStruct((2, _NPAD, d), jnp.float32),
        mesh=mesh,
        scratch_types=[
            pltpu.VMEM((_CHUNKS, _K), jnp.int32),
            pltpu.VMEM((_CHUNKS, _K), jnp.int32),
          ] + [pltpu.VMEM((_K, d), jnp.float32)] * _NB
          + [pltpu.VMEM_SHARED((_NPAD, d), jnp.float32)]
          + ([pltpu.VMEM_SHARED((_N, d), jnp.float32)] if d <= 32 else [])
          + [pltpu.SemaphoreType.DMA] * (2 * _NB),
        compiler_params=pltpu.CompilerParams(use_tc_tiling_on_sc=False),
    )
    def segsum(sup_hbm, src_hbm, dst_hbm, zero_hbm, out_hbm,
               src_v, dst_v, *bufs):
        rows = bufs[:_NB]
        acc_sh = bufs[_NB]
        if d <= 32:
            stage_sh = bufs[_NB + 1]
            gsems = bufs[_NB + 2:2 * _NB + 2]
        else:
            stage_sh = None
            gsems = bufs[_NB + 1:2 * _NB + 1]
        cid = lax.axis_index("c")
        sid = lax.axis_index("s")
        tile = cid * 16 + sid
        r0 = sid * _RPT
        pltpu.sync_copy(zero_hbm.at[pl.ds(r0, _RPT)],
                        acc_sh.at[pl.ds(r0, _RPT)])
        if stage_sh is not None:
            s0 = sid * (_N // 16)
            pltpu.sync_copy(sup_hbm.at[pl.ds(s0, _N // 16)],
                            stage_sh.at[pl.ds(s0, _N // 16)])
        gsrc = sup_hbm if stage_sh is None else stage_sh
        c0 = tile * _CHUNKS
        pltpu.sync_copy(src_hbm.at[pl.ds(c0, _CHUNKS)], src_v)
        pltpu.sync_copy(dst_hbm.at[pl.ds(c0, _CHUNKS)], dst_v)
        plsc.subcore_barrier()

        nb = _NB
        for b in range(nb):
            pltpu.async_copy(gsrc.at[src_v.at[b]], rows[b], gsems[b])

        def body(g, carry):
            for b in range(nb):
                i = g * nb + b
                pltpu.make_async_copy(gsrc.at[src_v.at[i]],
                                      rows[b], gsems[b]).wait()
                pltpu.sync_copy(rows[b], acc_sh.at[dst_v.at[i]], add=True)
                nxt = i + nb

                @pl.when(nxt < _CHUNKS)
                def _():
                    pltpu.async_copy(gsrc.at[src_v.at[nxt]],
                                     rows[b], gsems[b])
            return carry

        lax.fori_loop(0, _CHUNKS // nb, body, 0)
        plsc.subcore_barrier()
        pltpu.sync_copy(acc_sh.at[pl.ds(r0, _RPT)],
                        out_hbm.at[cid, pl.ds(r0, _RPT)])

    return segsum


_segsum64 = _make_segsum(64)
_segsum32 = _make_segsum(32)


def kernel(fea, edge_index, W1, b1, W2, b2, Wd1, bd1, Wd2, bd2):
    src = edge_index[0].reshape(_NTILES * _CHUNKS, _K)
    dst = edge_index[1].reshape(_NTILES * _CHUNKS, _K)
    zero64 = jnp.zeros((_NPAD, 64), jnp.float32)
    zero32 = jnp.zeros((_NPAD, 32), jnp.float32)

    sup1 = _mm1(fea, W1, b1.reshape(1, -1))
    p1 = _segsum64(sup1, src, dst, zero64)
    sup2 = _mm2(p1, W2, b2.reshape(1, -1))
    p2 = _segsum32(sup2, src, dst, zero32)
    recon, adj = _decode(p2, Wd1, bd1.reshape(1, -1), Wd2, bd2.reshape(1, -1))
    return recon, adj


# speedup vs baseline: 1.0118x; 1.0116x over previous
import functools

import jax
import jax.numpy as jnp
from jax import lax
from jax.experimental import pallas as pl
from jax.experimental.pallas import tpu as pltpu
from jax.experimental.pallas import tpu_sc as plsc

_N = 10000
_NPAD = 10240
_E = 320000
_NTILES = 32
_K = 125
_EPT = _E // _NTILES
_CHUNKS = _EPT // _K
_RPT = _NPAD // 16
_RDEC = 200
_NB = 8


def _mm1_body(x_ref, w_ref, b_ref, o_ref):
    o_ref[...] = (
        jnp.dot(x_ref[...], w_ref[...], preferred_element_type=jnp.float32)
        + b_ref[...]
    )


def _mm1(x, w, b):
    n, din = x.shape
    dout = w.shape[1]
    blk = 2000
    return pl.pallas_call(
        _mm1_body,
        grid=(n // blk,),
        in_specs=[
            pl.BlockSpec((blk, din), lambda i: (i, 0)),
            pl.BlockSpec((din, dout), lambda i: (0, 0)),
            pl.BlockSpec((1, dout), lambda i: (0, 0)),
        ],
        out_specs=pl.BlockSpec((blk, dout), lambda i: (i, 0)),
        out_shape=jax.ShapeDtypeStruct((n, dout), jnp.float32),
    )(x, w, b)


def _mm2_body(p_ref, w_ref, b_ref, o_ref):
    h = jnp.maximum(p_ref[0] + p_ref[1], 0.0)
    o_ref[...] = (
        jnp.dot(h, w_ref[...], preferred_element_type=jnp.float32) + b_ref[...]
    )


def _mm2(p, w, b):
    n = p.shape[1]
    din = p.shape[2]
    dout = w.shape[1]
    blk = 2048
    return pl.pallas_call(
        _mm2_body,
        grid=(n // blk,),
        in_specs=[
            pl.BlockSpec((2, blk, din), lambda i: (0, i, 0)),
            pl.BlockSpec((din, dout), lambda i: (0, 0)),
            pl.BlockSpec((1, dout), lambda i: (0, 0)),
        ],
        out_specs=pl.BlockSpec((blk, dout), lambda i: (i, 0)),
        out_shape=jax.ShapeDtypeStruct((n, dout), jnp.float32),
    )(p, w, b)


def _dec_body(zp_ref, zpb_ref, wd1_ref, bd1_ref, wd2_ref, bd2_ref,
              recon_ref, adj_ref):
    zfull = (zp_ref[0] + zp_ref[1])[:_N]
    zblk = zpb_ref[0] + zpb_ref[1]
    d = jnp.maximum(
        jnp.dot(zblk, wd1_ref[...], preferred_element_type=jnp.float32)
        + bd1_ref[...], 0.0)
    recon_ref[...] = jnp.maximum(
        jnp.dot(d, wd2_ref[...], preferred_element_type=jnp.float32)
        + bd2_ref[...], 0.0)
    logits = lax.dot_general(zblk, zfull, (((1,), (1,)), ((), ())),
                             preferred_element_type=jnp.float32)
    adj_ref[...] = 0.5 * jnp.tanh(0.5 * logits) + 0.5


def _decode(zp, wd1, bd1, wd2, bd2):
    n = _N
    npad = zp.shape[1]
    dz = zp.shape[2]
    d1 = wd1.shape[1]
    d0 = wd2.shape[1]
    grid = (n // _RDEC,)
    return pl.pallas_call(
        _dec_body,
        grid=grid,
        in_specs=[
            pl.BlockSpec((2, npad, dz), lambda i: (0, 0, 0)),
            pl.BlockSpec((2, _RDEC, dz), lambda i: (0, i, 0)),
            pl.BlockSpec((dz, d1), lambda i: (0, 0)),
            pl.BlockSpec((1, d1), lambda i: (0, 0)),
            pl.BlockSpec((d1, d0), lambda i: (0, 0)),
            pl.BlockSpec((1, d0), lambda i: (0, 0)),
        ],
        out_specs=[
            pl.BlockSpec((_RDEC, d0), lambda i: (i, 0)),
            pl.BlockSpec((_RDEC, n), lambda i: (i, 0)),
        ],
        out_shape=[
            jax.ShapeDtypeStruct((n, d0), jnp.float32),
            jax.ShapeDtypeStruct((n, n), jnp.float32),
        ],
    )(zp, zp, wd1, bd1, wd2, bd2)


def _make_segsum(d):
    mesh = plsc.VectorSubcoreMesh(core_axis_name="c", subcore_axis_name="s")

    @functools.partial(
        pl.kernel,
        out_type=jax.ShapeDtypeStruct((2, _NPAD, d), jnp.float32),
        mesh=mesh,
        scratch_types=[
            pltpu.VMEM((_CHUNKS, _K), jnp.int32),
            pltpu.VMEM((_CHUNKS, _K), jnp.int32),
          ] + [pltpu.VMEM((_K, d), jnp.float32)] * _NB
          + [pltpu.VMEM_SHARED((_NPAD, d), jnp.float32)]
          + [pltpu.SemaphoreType.DMA] * _NB,
        compiler_params=pltpu.CompilerParams(use_tc_tiling_on_sc=False),
    )
    def segsum(sup_hbm, src_hbm, dst_hbm, zero_hbm, out_hbm,
               src_v, dst_v, *bufs):
        rows = bufs[:_NB]
        acc_sh = bufs[_NB]
        gsems = bufs[_NB + 1:]
        cid = lax.axis_index("c")
        sid = lax.axis_index("s")
        tile = cid * 16 + sid
        r0 = sid * _RPT
        pltpu.sync_copy(zero_hbm.at[pl.ds(r0, _RPT)],
                        acc_sh.at[pl.ds(r0, _RPT)])
        gsrc = sup_hbm
        c0 = tile * _CHUNKS
        pltpu.sync_copy(src_hbm.at[pl.ds(c0, _CHUNKS)], src_v)
        pltpu.sync_copy(dst_hbm.at[pl.ds(c0, _CHUNKS)], dst_v)
        plsc.subcore_barrier()

        nb = _NB
        for b in range(nb):
            pltpu.async_copy(gsrc.at[src_v.at[b]], rows[b], gsems[b])

        def body(g, carry):
            for b in range(nb):
                i = g * nb + b
                pltpu.make_async_copy(gsrc.at[src_v.at[i]],
                                      rows[b], gsems[b]).wait()
                pltpu.sync_copy(rows[b], acc_sh.at[dst_v.at[i]], add=True)
                nxt = i + nb

                @pl.when(nxt < _CHUNKS)
                def _():
                    pltpu.async_copy(gsrc.at[src_v.at[nxt]],
                                     rows[b], gsems[b])
            return carry

        lax.fori_loop(0, _CHUNKS // nb, body, 0)
        plsc.subcore_barrier()
        pltpu.sync_copy(acc_sh.at[pl.ds(r0, _RPT)],
                        out_hbm.at[cid, pl.ds(r0, _RPT)])

    return segsum


_segsum64 = _make_segsum(64)
_segsum32 = _make_segsum(32)


def kernel(fea, edge_index, W1, b1, W2, b2, Wd1, bd1, Wd2, bd2):
    src = edge_index[0].reshape(_NTILES * _CHUNKS, _K)
    dst = edge_index[1].reshape(_NTILES * _CHUNKS, _K)
    zero64 = jnp.zeros((_NPAD, 64), jnp.float32)
    zero32 = jnp.zeros((_NPAD, 32), jnp.float32)

    sup1 = _mm1(fea, W1, b1.reshape(1, -1))
    p1 = _segsum64(sup1, src, dst, zero64)
    sup2 = _mm2(p1, W2, b2.reshape(1, -1))
    p2 = _segsum32(sup2, src, dst, zero32)
    recon, adj = _decode(p2, Wd1, bd1.reshape(1, -1), Wd2, bd2.reshape(1, -1))
    return recon, adj


# speedup vs baseline: 1.0144x; 1.0026x over previous
import functools

import jax
import jax.numpy as jnp
from jax import lax
from jax.experimental import pallas as pl
from jax.experimental.pallas import tpu as pltpu
from jax.experimental.pallas import tpu_sc as plsc

_N = 10000
_NPAD = 10240
_E = 320000
_NTILES = 32
_K = 125
_EPT = _E // _NTILES
_CHUNKS = _EPT // _K
_RPT = _NPAD // 16
_RDEC = 400
_NB = 8


def _mm1_body(x_ref, w_ref, b_ref, o_ref):
    o_ref[...] = (
        jnp.dot(x_ref[...], w_ref[...], preferred_element_type=jnp.float32)
        + b_ref[...]
    )


def _mm1(x, w, b):
    n, din = x.shape
    dout = w.shape[1]
    blk = 2000
    return pl.pallas_call(
        _mm1_body,
        grid=(n // blk,),
        in_specs=[
            pl.BlockSpec((blk, din), lambda i: (i, 0)),
            pl.BlockSpec((din, dout), lambda i: (0, 0)),
            pl.BlockSpec((1, dout), lambda i: (0, 0)),
        ],
        out_specs=pl.BlockSpec((blk, dout), lambda i: (i, 0)),
        out_shape=jax.ShapeDtypeStruct((n, dout), jnp.float32),
    )(x, w, b)


def _mm2_body(p_ref, w_ref, b_ref, o_ref):
    h = jnp.maximum(p_ref[0] + p_ref[1], 0.0)
    o_ref[...] = (
        jnp.dot(h, w_ref[...], preferred_element_type=jnp.float32) + b_ref[...]
    )


def _mm2(p, w, b):
    n = p.shape[1]
    din = p.shape[2]
    dout = w.shape[1]
    blk = 2048
    return pl.pallas_call(
        _mm2_body,
        grid=(n // blk,),
        in_specs=[
            pl.BlockSpec((2, blk, din), lambda i: (0, i, 0)),
            pl.BlockSpec((din, dout), lambda i: (0, 0)),
            pl.BlockSpec((1, dout), lambda i: (0, 0)),
        ],
        out_specs=pl.BlockSpec((blk, dout), lambda i: (i, 0)),
        out_shape=jax.ShapeDtypeStruct((n, dout), jnp.float32),
    )(p, w, b)


def _dec_body(zp_ref, zpb_ref, wd1_ref, bd1_ref, wd2_ref, bd2_ref,
              recon_ref, adj_ref):
    zfull = (zp_ref[0] + zp_ref[1])[:_N]
    zblk = zpb_ref[0] + zpb_ref[1]
    d = jnp.maximum(
        jnp.dot(zblk, wd1_ref[...], preferred_element_type=jnp.float32)
        + bd1_ref[...], 0.0)
    recon_ref[...] = jnp.maximum(
        jnp.dot(d, wd2_ref[...], preferred_element_type=jnp.float32)
        + bd2_ref[...], 0.0)
    logits = lax.dot_general(zblk, zfull, (((1,), (1,)), ((), ())),
                             preferred_element_type=jnp.float32)
    adj_ref[...] = 0.5 * jnp.tanh(0.5 * logits) + 0.5


def _decode(zp, wd1, bd1, wd2, bd2):
    n = _N
    npad = zp.shape[1]
    dz = zp.shape[2]
    d1 = wd1.shape[1]
    d0 = wd2.shape[1]
    grid = (n // _RDEC,)
    return pl.pallas_call(
        _dec_body,
        grid=grid,
        in_specs=[
            pl.BlockSpec((2, npad, dz), lambda i: (0, 0, 0)),
            pl.BlockSpec((2, _RDEC, dz), lambda i: (0, i, 0)),
            pl.BlockSpec((dz, d1), lambda i: (0, 0)),
            pl.BlockSpec((1, d1), lambda i: (0, 0)),
            pl.BlockSpec((d1, d0), lambda i: (0, 0)),
            pl.BlockSpec((1, d0), lambda i: (0, 0)),
        ],
        out_specs=[
            pl.BlockSpec((_RDEC, d0), lambda i: (i, 0)),
            pl.BlockSpec((_RDEC, n), lambda i: (i, 0)),
        ],
        out_shape=[
            jax.ShapeDtypeStruct((n, d0), jnp.float32),
            jax.ShapeDtypeStruct((n, n), jnp.float32),
        ],
    )(zp, zp, wd1, bd1, wd2, bd2)


def _make_segsum(d):
    mesh = plsc.VectorSubcoreMesh(core_axis_name="c", subcore_axis_name="s")

    @functools.partial(
        pl.kernel,
        out_type=jax.ShapeDtypeStruct((2, _NPAD, d), jnp.float32),
        mesh=mesh,
        scratch_types=[
            pltpu.VMEM((_CHUNKS, _K), jnp.int32),
            pltpu.VMEM((_CHUNKS, _K), jnp.int32),
          ] + [pltpu.VMEM((_K, d), jnp.float32)] * _NB
          + [pltpu.VMEM_SHARED((_NPAD, d), jnp.float32)]
          + [pltpu.SemaphoreType.DMA] * _NB,
        compiler_params=pltpu.CompilerParams(use_tc_tiling_on_sc=False),
    )
    def segsum(sup_hbm, src_hbm, dst_hbm, zero_hbm, out_hbm,
               src_v, dst_v, *bufs):
        rows = bufs[:_NB]
        acc_sh = bufs[_NB]
        gsems = bufs[_NB + 1:]
        cid = lax.axis_index("c")
        sid = lax.axis_index("s")
        tile = cid * 16 + sid
        r0 = sid * _RPT
        pltpu.sync_copy(zero_hbm.at[pl.ds(r0, _RPT)],
                        acc_sh.at[pl.ds(r0, _RPT)])
        gsrc = sup_hbm
        c0 = tile * _CHUNKS
        pltpu.sync_copy(src_hbm.at[pl.ds(c0, _CHUNKS)], src_v)
        pltpu.sync_copy(dst_hbm.at[pl.ds(c0, _CHUNKS)], dst_v)
        plsc.subcore_barrier()

        nb = _NB
        for b in range(nb):
            pltpu.async_copy(gsrc.at[src_v.at[b]], rows[b], gsems[b])

        def body(g, carry):
            for b in range(nb):
                i = g * nb + b
                pltpu.make_async_copy(gsrc.at[src_v.at[i]],
                                      rows[b], gsems[b]).wait()
                pltpu.sync_copy(rows[b], acc_sh.at[dst_v.at[i]], add=True)
                nxt = i + nb

                @pl.when(nxt < _CHUNKS)
                def _():
                    pltpu.async_copy(gsrc.at[src_v.at[nxt]],
                                     rows[b], gsems[b])
            return carry

        lax.fori_loop(0, _CHUNKS // nb, body, 0)
        plsc.subcore_barrier()
        pltpu.sync_copy(acc_sh.at[pl.ds(r0, _RPT)],
                        out_hbm.at[cid, pl.ds(r0, _RPT)])

    return segsum


_segsum64 = _make_segsum(64)
_segsum32 = _make_segsum(32)


def kernel(fea, edge_index, W1, b1, W2, b2, Wd1, bd1, Wd2, bd2):
    src = edge_index[0].reshape(_NTILES * _CHUNKS, _K)
    dst = edge_index[1].reshape(_NTILES * _CHUNKS, _K)
    zero64 = jnp.zeros((_NPAD, 64), jnp.float32)
    zero32 = jnp.zeros((_NPAD, 32), jnp.float32)

    sup1 = _mm1(fea, W1, b1.reshape(1, -1))
    p1 = _segsum64(sup1, src, dst, zero64)
    sup2 = _mm2(p1, W2, b2.reshape(1, -1))
    p2 = _segsum32(sup2, src, dst, zero32)
    recon, adj = _decode(p2, Wd1, bd1.reshape(1, -1), Wd2, bd2.reshape(1, -1))
    return recon, adj


# speedup vs baseline: 1.0414x; 1.0266x over previous
import functools

import jax
import jax.numpy as jnp
from jax import lax
from jax.experimental import pallas as pl
from jax.experimental.pallas import tpu as pltpu
from jax.experimental.pallas import tpu_sc as plsc

_N = 10000
_NPAD = 10240
_E = 320000
_NTILES = 32
_K = 125
_EPT = _E // _NTILES
_CHUNKS = _EPT // _K
_RPT = _NPAD // 16
_RDEC = 400
_NB = 8


def _mm1_body(x_ref, w_ref, b_ref, o_ref):
    o_ref[...] = (
        jnp.dot(x_ref[...], w_ref[...], preferred_element_type=jnp.float32)
        + b_ref[...]
    )


def _mm1(x, w, b):
    n, din = x.shape
    dout = w.shape[1]
    return pl.pallas_call(
        _mm1_body,
        out_shape=jax.ShapeDtypeStruct((n, dout), jnp.float32),
    )(x, w, b)


def _mm2_body(p_ref, w_ref, b_ref, o_ref):
    h = jnp.maximum(p_ref[0] + p_ref[1], 0.0)
    o_ref[...] = (
        jnp.dot(h, w_ref[...], preferred_element_type=jnp.float32) + b_ref[...]
    )


def _mm2(p, w, b):
    n = p.shape[1]
    dout = w.shape[1]
    return pl.pallas_call(
        _mm2_body,
        out_shape=jax.ShapeDtypeStruct((n, dout), jnp.float32),
    )(p, w, b)


def _dec_body(zp_ref, zpb_ref, wd1_ref, bd1_ref, wd2_ref, bd2_ref,
              recon_ref, adj_ref):
    zfull = (zp_ref[0] + zp_ref[1])[:_N]
    zblk = zpb_ref[0] + zpb_ref[1]
    d = jnp.maximum(
        jnp.dot(zblk, wd1_ref[...], preferred_element_type=jnp.float32)
        + bd1_ref[...], 0.0)
    recon_ref[...] = jnp.maximum(
        jnp.dot(d, wd2_ref[...], preferred_element_type=jnp.float32)
        + bd2_ref[...], 0.0)
    logits = lax.dot_general(zblk, zfull, (((1,), (1,)), ((), ())),
                             preferred_element_type=jnp.float32)
    adj_ref[...] = 0.5 * jnp.tanh(0.5 * logits) + 0.5


def _decode(zp, wd1, bd1, wd2, bd2):
    n = _N
    npad = zp.shape[1]
    dz = zp.shape[2]
    d1 = wd1.shape[1]
    d0 = wd2.shape[1]
    grid = (n // _RDEC,)
    return pl.pallas_call(
        _dec_body,
        grid=grid,
        in_specs=[
            pl.BlockSpec((2, npad, dz), lambda i: (0, 0, 0)),
            pl.BlockSpec((2, _RDEC, dz), lambda i: (0, i, 0)),
            pl.BlockSpec((dz, d1), lambda i: (0, 0)),
            pl.BlockSpec((1, d1), lambda i: (0, 0)),
            pl.BlockSpec((d1, d0), lambda i: (0, 0)),
            pl.BlockSpec((1, d0), lambda i: (0, 0)),
        ],
        out_specs=[
            pl.BlockSpec((_RDEC, d0), lambda i: (i, 0)),
            pl.BlockSpec((_RDEC, n), lambda i: (i, 0)),
        ],
        out_shape=[
            jax.ShapeDtypeStruct((n, d0), jnp.float32),
            jax.ShapeDtypeStruct((n, n), jnp.float32),
        ],
    )(zp, zp, wd1, bd1, wd2, bd2)


def _make_segsum(d):
    mesh = plsc.VectorSubcoreMesh(core_axis_name="c", subcore_axis_name="s")

    @functools.partial(
        pl.kernel,
        out_type=jax.ShapeDtypeStruct((2, _NPAD, d), jnp.float32),
        mesh=mesh,
        scratch_types=[
            pltpu.VMEM((_CHUNKS, _K), jnp.int32),
            pltpu.VMEM((_CHUNKS, _K), jnp.int32),
          ] + [pltpu.VMEM((_K, d), jnp.float32)] * _NB
          + [pltpu.VMEM_SHARED((_NPAD, d), jnp.float32)]
          + [pltpu.SemaphoreType.DMA] * (_NB + 3),
        compiler_params=pltpu.CompilerParams(use_tc_tiling_on_sc=False),
    )
    def segsum(sup_hbm, src_hbm, dst_hbm, zero_hbm, out_hbm,
               src_v, dst_v, *bufs):
        rows = bufs[:_NB]
        acc_sh = bufs[_NB]
        gsems = bufs[_NB + 1:2 * _NB + 1]
        psems = bufs[2 * _NB + 1:]
        cid = lax.axis_index("c")
        sid = lax.axis_index("s")
        tile = cid * 16 + sid
        r0 = sid * _RPT
        gsrc = sup_hbm
        c0 = tile * _CHUNKS
        pltpu.async_copy(zero_hbm.at[pl.ds(r0, _RPT)],
                         acc_sh.at[pl.ds(r0, _RPT)], psems[0])
        pltpu.async_copy(src_hbm.at[pl.ds(c0, _CHUNKS)], src_v, psems[1])
        pltpu.async_copy(dst_hbm.at[pl.ds(c0, _CHUNKS)], dst_v, psems[2])
        pltpu.make_async_copy(src_hbm.at[pl.ds(c0, _CHUNKS)], src_v,
                              psems[1]).wait()
        pltpu.make_async_copy(dst_hbm.at[pl.ds(c0, _CHUNKS)], dst_v,
                              psems[2]).wait()

        nb = _NB
        for b in range(nb):
            pltpu.async_copy(gsrc.at[src_v.at[b]], rows[b], gsems[b])

        pltpu.make_async_copy(zero_hbm.at[pl.ds(r0, _RPT)],
                              acc_sh.at[pl.ds(r0, _RPT)], psems[0]).wait()
        plsc.subcore_barrier()

        def body(g, carry):
            for b in range(nb):
                i = g * nb + b
                pltpu.make_async_copy(gsrc.at[src_v.at[i]],
                                      rows[b], gsems[b]).wait()
                pltpu.sync_copy(rows[b], acc_sh.at[dst_v.at[i]], add=True)
                nxt = i + nb

                @pl.when(nxt < _CHUNKS)
                def _():
                    pltpu.async_copy(gsrc.at[src_v.at[nxt]],
                                     rows[b], gsems[b])
            return carry

        lax.fori_loop(0, _CHUNKS // nb, body, 0)
        plsc.subcore_barrier()
        pltpu.sync_copy(acc_sh.at[pl.ds(r0, _RPT)],
                        out_hbm.at[cid, pl.ds(r0, _RPT)])

    return segsum


_segsum64 = _make_segsum(64)
_segsum32 = _make_segsum(32)


def kernel(fea, edge_index, W1, b1, W2, b2, Wd1, bd1, Wd2, bd2):
    src = edge_index[0].reshape(_NTILES * _CHUNKS, _K)
    dst = edge_index[1].reshape(_NTILES * _CHUNKS, _K)
    zero64 = jnp.zeros((_NPAD, 64), jnp.float32)
    zero32 = jnp.zeros((_NPAD, 32), jnp.float32)

    sup1 = _mm1(fea, W1, b1.reshape(1, -1))
    p1 = _segsum64(sup1, src, dst, zero64)
    sup2 = _mm2(p1, W2, b2.reshape(1, -1))
    p2 = _segsum32(sup2, src, dst, zero32)
    recon, adj = _decode(p2, Wd1, bd1.reshape(1, -1), Wd2, bd2.reshape(1, -1))
    return recon, adj
